# baseline (speedup 1.0000x reference)
import functools

import jax
import jax.numpy as jnp
from jax import lax
from jax.experimental import pallas as pl
from jax.experimental.pallas import tpu as pltpu
from jax.experimental.pallas import tpu_sc as plsc

N = 10000
E = 320000
DIN = 128
DH = 128
DOUT = 40
D3 = 128
BN_EPS = 1e-5

NC = 2
NS = 16
NW = NC * NS
CHUNK = 128
GEC = 8
NBUF = 2
NPAD = 10240
EWP = NPAD
NCH = EWP // CHUNK
NGRP = NCH // GEC
NSUP = NGRP // 2
RPT = NPAD // NS

ADT = jnp.float32


def _sc_agg_body(d, t_hbm, src_hbm, dst_hbm, out_hbm,
                 sring, dring, bufs, acc, isems, dsems, gsems):
    cid = lax.axis_index("c")
    sid = lax.axis_index("s")
    wid = sid * NC + cid

    lanes = 32 if ADT == jnp.bfloat16 else 16

    def zrow(r, carry):
        for k in range(d // lanes):
            bufs[0][r, pl.ds(k * lanes, lanes)] = jnp.zeros((lanes,), ADT)
        return carry

    lax.fori_loop(0, CHUNK, zrow, 0)
    for i in range(RPT // CHUNK):
        pltpu.sync_copy(bufs[0],
                        acc.at[pl.ds(sid * RPT + i * CHUNK, CHUNK)])
    plsc.subcore_barrier()

    def ldsrc(g, s):
        return pltpu.make_async_copy(src_hbm.at[wid, g], sring[s], isems[s])

    def lddst(g, s):
        return pltpu.make_async_copy(dst_hbm.at[wid, g], dring[s], dsems[s])

    def gth(s, t, b):
        return pltpu.make_async_copy(t_hbm.at[sring[s].at[t]], bufs[b],
                                     gsems[b])

    def sct(s, t, b):
        pltpu.sync_copy(bufs[b], acc.at[dring[s].at[t]], add=True)

    for s in range(2):
        ldsrc(s, s).start()
        lddst(s, s).start()
    ldsrc(0, 0).wait()
    for b in range(NBUF):
        gth(0, b, b).start()
    lddst(0, 0).wait()

    def super_body(k, carry):
        g_next0 = jnp.minimum(2 * k + 2, NGRP - 1)
        g_next1 = jnp.minimum(2 * k + 3, NGRP - 1)
        for half, s in ((0, 0), (1, 1)):
            ns = 1 - s
            for t in range(GEC):
                b = t % NBUF
                gth(s, t, b).wait()
                sct(s, t, b)
                if t < GEC - NBUF:
                    gth(s, t + NBUF, b).start()
                else:
                    if t == GEC - NBUF:
                        ldsrc(0, ns).wait()
                    gth(ns, t - (GEC - NBUF), b).start()
            if half == 0:
                ldsrc(g_next0, 0).start()
                lddst(g_next0, 0).start()
                lddst(1, 1).wait()
        ldsrc(g_next1, 1).start()
        lddst(g_next1, 1).start()
        lddst(0, 0).wait()
        return carry

    lax.fori_loop(0, NSUP, super_body, 0)
    for b in range(NBUF):
        pltpu.make_async_copy(t_hbm.at[sring[0].at[0]], bufs[b],
                              gsems[b]).wait()
    ldsrc(0, 1).wait()
    lddst(0, 1).wait()
    plsc.subcore_barrier()

    pltpu.sync_copy(acc.at[pl.ds(sid * RPT, RPT)],
                    out_hbm.at[cid, pl.ds(sid * RPT, RPT)])


def _make_sc_agg(d):
    mesh = plsc.VectorSubcoreMesh(core_axis_name="c", subcore_axis_name="s",
                                  num_cores=NC, num_subcores=NS)
    return pl.kernel(
        functools.partial(_sc_agg_body, d),
        out_type=jax.ShapeDtypeStruct((NC, NPAD, d), ADT),
        mesh=mesh,
        scratch_types=[
            [pltpu.VMEM((GEC, CHUNK), jnp.int32) for _ in range(2)],
            [pltpu.VMEM((GEC, CHUNK), jnp.int32) for _ in range(2)],
            [pltpu.VMEM((CHUNK, d), ADT) for _ in range(NBUF)],
            pltpu.VMEM_SHARED((NPAD, d), ADT),
            [pltpu.SemaphoreType.DMA for _ in range(2)],
            [pltpu.SemaphoreType.DMA for _ in range(2)],
            [pltpu.SemaphoreType.DMA for _ in range(NBUF)],
        ],
    )


BN_ROWS = 400


def _mm_body(x_ref, w_ref, o_ref):
    o_ref[...] = jnp.dot(x_ref[...], w_ref[...],
                         preferred_element_type=jnp.float32)


def _tc_matmul(x, w, dout):
    return pl.pallas_call(
        _mm_body,
        grid=(N // BN_ROWS,),
        in_specs=[
            pl.BlockSpec((BN_ROWS, x.shape[1]), lambda i: (i, 0)),
            pl.BlockSpec(w.shape, lambda i: (0, 0)),
        ],
        out_specs=pl.BlockSpec((BN_ROWS, dout), lambda i: (i, 0)),
        out_shape=jax.ShapeDtypeStruct((N, dout), jnp.float32),
    )(x, w)


def _stage_body(t_ref, p0_ref, p1_ref, eps_ref, b_ref, a_ref, be_ref, w_ref,
                o_ref):
    agg = p0_ref[0] + p1_ref[0]
    z = (1.0 + eps_ref[0, 0]) * t_ref[...] + agg + b_ref[...]
    h = jnp.maximum(z * a_ref[...] + be_ref[...], 0.0)
    o_ref[...] = jnp.dot(h, w_ref[...], preferred_element_type=jnp.float32)


def _tc_stage(t, p, eps, b, a, be, w, dout):
    return pl.pallas_call(
        _stage_body,
        grid=(N // BN_ROWS,),
        in_specs=[
            pl.BlockSpec((BN_ROWS, DH), lambda i: (i, 0)),
            pl.BlockSpec((1, BN_ROWS, DH), lambda i: (0, i, 0)),
            pl.BlockSpec((1, BN_ROWS, DH), lambda i: (1, i, 0)),
            pl.BlockSpec(memory_space=pltpu.SMEM),
            pl.BlockSpec((1, DH), lambda i: (0, 0)),
            pl.BlockSpec((1, DH), lambda i: (0, 0)),
            pl.BlockSpec((1, DH), lambda i: (0, 0)),
            pl.BlockSpec((DH, dout), lambda i: (0, 0)),
        ],
        out_specs=pl.BlockSpec((BN_ROWS, dout), lambda i: (i, 0)),
        out_shape=jax.ShapeDtypeStruct((N, dout), jnp.float32),
    )(t, p, p, eps, b, a, be, w)


def _final_body(t_ref, p0_ref, p1_ref, eps_ref, b_ref, o_ref):
    agg = p0_ref[0] + p1_ref[0]
    z = (1.0 + eps_ref[0, 0]) * t_ref[...] + agg + b_ref[...]
    mask = lax.broadcasted_iota(jnp.int32, z.shape, 1) < DOUT
    zm = jnp.where(mask, z, -jnp.inf)
    m = jnp.max(zm, axis=-1, keepdims=True)
    ez = jnp.where(mask, jnp.exp(z - m), 0.0)
    ls = (z - m) - jnp.log(jnp.sum(ez, axis=-1, keepdims=True))
    o_ref[...] = ls[:, :DOUT]


def _tc_final(t, p, eps, b, d):
    return pl.pallas_call(
        _final_body,
        grid=(N // BN_ROWS,),
        in_specs=[
            pl.BlockSpec((BN_ROWS, d), lambda i: (i, 0)),
            pl.BlockSpec((1, BN_ROWS, d), lambda i: (0, i, 0)),
            pl.BlockSpec((1, BN_ROWS, d), lambda i: (1, i, 0)),
            pl.BlockSpec(memory_space=pltpu.SMEM),
            pl.BlockSpec((1, d), lambda i: (0, 0)),
        ],
        out_specs=pl.BlockSpec((BN_ROWS, DOUT), lambda i: (i, 0)),
        out_shape=jax.ShapeDtypeStruct((N, DOUT), jnp.float32),
    )(t, p, p, eps, b)


@jax.jit
def _run(x, edge_index, W0, b0, W1, b1, W2, b2, eps0, eps1, eps2,
         g0, be0, g1, be1):
    npad_e = NW * EWP - E
    pad_src = jnp.arange(npad_e, dtype=jnp.int32) % N
    pad_dst = N + (jnp.arange(npad_e, dtype=jnp.int32) % (NPAD - N))
    src3 = jnp.concatenate([edge_index[0], pad_src]).reshape(
        NW, NGRP, GEC, CHUNK)
    dst3 = jnp.concatenate([edge_index[1], pad_dst]).reshape(
        NW, NGRP, GEC, CHUNK)

    bn_s = 1.0 / jnp.sqrt(1.0 + BN_EPS)
    a0 = (g0 * bn_s).reshape(1, DH)
    a1 = (g1 * bn_s).reshape(1, DH)

    sc_agg = _make_sc_agg(DH)
    sc_agg3 = _make_sc_agg(D3)

    t0 = _tc_matmul(x, W0, DH)
    p0 = sc_agg(t0, src3, dst3)
    t1 = _tc_stage(t0, p0, eps0.reshape(1, 1), b0.reshape(1, DH), a0,
                   be0.reshape(1, DH), W1, DH)
    p1 = sc_agg(t1, src3, dst3)
    t2 = _tc_stage(t1, p1, eps1.reshape(1, 1), b1.reshape(1, DH), a1,
                   be1.reshape(1, DH),
                   jnp.pad(W2, ((0, 0), (0, D3 - DOUT))), D3)
    p2 = sc_agg3(t2, src3, dst3)
    b2p = jnp.pad(b2.reshape(1, DOUT), ((0, 0), (0, D3 - DOUT)))
    return _tc_final(t2, p2, eps2.reshape(1, 1), b2p, D3)


def kernel(x, edge_index, W0, b0, W1, b1, W2, b2, eps0, eps1, eps2,
           g0, be0, g1, be1):
    return _run(x, edge_index, W0, b0, W1, b1, W2, b2, eps0, eps1, eps2,
                g0, be0, g1, be1)


# speedup vs baseline: 1.0959x; 1.0959x over previous
import functools

import jax
import jax.numpy as jnp
from jax import lax
from jax.experimental import pallas as pl
from jax.experimental.pallas import tpu as pltpu
from jax.experimental.pallas import tpu_sc as plsc

N = 10000
E = 320000
DIN = 128
DH = 128
DOUT = 40
D3 = 128
BN_EPS = 1e-5

NC = 2
NS = 16
NW = NC * NS
CHUNK = 128
GEC = 8
NBUF = 2
NPAD = 10240
EWP = NPAD
NCH = EWP // CHUNK
NGRP = NCH // GEC
NSUP = NGRP // 2
RPT = NPAD // NS

ADT = jnp.float32


def _sc_agg_body(d, t_hbm, src_hbm, dst_hbm, out_hbm,
                 sring, dring, bufs, acc, isems, dsems, gsems):
    cid = lax.axis_index("c")
    sid = lax.axis_index("s")
    wid = sid * NC + cid

    lanes = 32 if ADT == jnp.bfloat16 else 16

    def zrow(r, carry):
        for k in range(d // lanes):
            bufs[0][r, pl.ds(k * lanes, lanes)] = jnp.zeros((lanes,), ADT)
        return carry

    lax.fori_loop(0, CHUNK, zrow, 0)
    for i in range(RPT // CHUNK):
        pltpu.sync_copy(bufs[0],
                        acc.at[pl.ds(sid * RPT + i * CHUNK, CHUNK)])
    plsc.subcore_barrier()

    def ldsrc(g, s):
        return pltpu.make_async_copy(src_hbm.at[wid, g], sring[s], isems[s])

    def lddst(g, s):
        return pltpu.make_async_copy(dst_hbm.at[wid, g], dring[s], dsems[s])

    def gth(s, t, b):
        return pltpu.make_async_copy(t_hbm.at[sring[s].at[t]], bufs[b],
                                     gsems[b])

    def sct(s, t, b):
        pltpu.sync_copy(bufs[b], acc.at[dring[s].at[t]], add=True)

    for s in range(2):
        ldsrc(s, s).start()
        lddst(s, s).start()
    ldsrc(0, 0).wait()
    for b in range(NBUF):
        gth(0, b, b).start()
    lddst(0, 0).wait()

    def super_body(k, carry):
        g_next0 = jnp.minimum(2 * k + 2, NGRP - 1)
        g_next1 = jnp.minimum(2 * k + 3, NGRP - 1)
        for half, s in ((0, 0), (1, 1)):
            ns = 1 - s
            for t in range(GEC):
                b = t % NBUF
                gth(s, t, b).wait()
                sct(s, t, b)
                if t < GEC - NBUF:
                    gth(s, t + NBUF, b).start()
                else:
                    if t == GEC - NBUF:
                        ldsrc(0, ns).wait()
                    gth(ns, t - (GEC - NBUF), b).start()
            if half == 0:
                ldsrc(g_next0, 0).start()
                lddst(g_next0, 0).start()
                lddst(1, 1).wait()
        ldsrc(g_next1, 1).start()
        lddst(g_next1, 1).start()
        lddst(0, 0).wait()
        return carry

    lax.fori_loop(0, NSUP, super_body, 0)
    for b in range(NBUF):
        pltpu.make_async_copy(t_hbm.at[sring[0].at[0]], bufs[b],
                              gsems[b]).wait()
    ldsrc(0, 1).wait()
    lddst(0, 1).wait()
    plsc.subcore_barrier()

    pltpu.sync_copy(acc.at[pl.ds(sid * RPT, RPT)],
                    out_hbm.at[cid, pl.ds(sid * RPT, RPT)])


def _make_sc_agg(d):
    mesh = plsc.VectorSubcoreMesh(core_axis_name="c", subcore_axis_name="s",
                                  num_cores=NC, num_subcores=NS)
    return pl.kernel(
        functools.partial(_sc_agg_body, d),
        out_type=jax.ShapeDtypeStruct((NC, NPAD, d), ADT),
        mesh=mesh,
        scratch_types=[
            [pltpu.VMEM((GEC, CHUNK), jnp.int32) for _ in range(2)],
            [pltpu.VMEM((GEC, CHUNK), jnp.int32) for _ in range(2)],
            [pltpu.VMEM((CHUNK, d), ADT) for _ in range(NBUF)],
            pltpu.VMEM_SHARED((NPAD, d), ADT),
            [pltpu.SemaphoreType.DMA for _ in range(2)],
            [pltpu.SemaphoreType.DMA for _ in range(2)],
            [pltpu.SemaphoreType.DMA for _ in range(NBUF)],
        ],
    )


BN_ROWS = 2000


def _mm_body(x_ref, w_ref, o_ref):
    o_ref[...] = jnp.dot(x_ref[...], w_ref[...],
                         preferred_element_type=jnp.float32)


def _tc_matmul(x, w, dout):
    return pl.pallas_call(
        _mm_body,
        grid=(N // BN_ROWS,),
        in_specs=[
            pl.BlockSpec((BN_ROWS, x.shape[1]), lambda i: (i, 0)),
            pl.BlockSpec(w.shape, lambda i: (0, 0)),
        ],
        out_specs=pl.BlockSpec((BN_ROWS, dout), lambda i: (i, 0)),
        out_shape=jax.ShapeDtypeStruct((N, dout), jnp.float32),
    )(x, w)


def _stage_body(t_ref, p0_ref, p1_ref, eps_ref, b_ref, a_ref, be_ref, w_ref,
                o_ref):
    agg = p0_ref[0] + p1_ref[0]
    z = (1.0 + eps_ref[0, 0]) * t_ref[...] + agg + b_ref[...]
    h = jnp.maximum(z * a_ref[...] + be_ref[...], 0.0)
    o_ref[...] = jnp.dot(h, w_ref[...], preferred_element_type=jnp.float32)


def _tc_stage(t, p, eps, b, a, be, w, dout):
    return pl.pallas_call(
        _stage_body,
        grid=(N // BN_ROWS,),
        in_specs=[
            pl.BlockSpec((BN_ROWS, DH), lambda i: (i, 0)),
            pl.BlockSpec((1, BN_ROWS, DH), lambda i: (0, i, 0)),
            pl.BlockSpec((1, BN_ROWS, DH), lambda i: (1, i, 0)),
            pl.BlockSpec(memory_space=pltpu.SMEM),
            pl.BlockSpec((1, DH), lambda i: (0, 0)),
            pl.BlockSpec((1, DH), lambda i: (0, 0)),
            pl.BlockSpec((1, DH), lambda i: (0, 0)),
            pl.BlockSpec((DH, dout), lambda i: (0, 0)),
        ],
        out_specs=pl.BlockSpec((BN_ROWS, dout), lambda i: (i, 0)),
        out_shape=jax.ShapeDtypeStruct((N, dout), jnp.float32),
    )(t, p, p, eps, b, a, be, w)


def _final_body(t_ref, p0_ref, p1_ref, eps_ref, b_ref, o_ref):
    agg = p0_ref[0] + p1_ref[0]
    z = (1.0 + eps_ref[0, 0]) * t_ref[...] + agg + b_ref[...]
    mask = lax.broadcasted_iota(jnp.int32, z.shape, 1) < DOUT
    zm = jnp.where(mask, z, -jnp.inf)
    m = jnp.max(zm, axis=-1, keepdims=True)
    ez = jnp.where(mask, jnp.exp(z - m), 0.0)
    ls = (z - m) - jnp.log(jnp.sum(ez, axis=-1, keepdims=True))
    o_ref[...] = ls[:, :DOUT]


def _tc_final(t, p, eps, b, d):
    return pl.pallas_call(
        _final_body,
        grid=(N // BN_ROWS,),
        in_specs=[
            pl.BlockSpec((BN_ROWS, d), lambda i: (i, 0)),
            pl.BlockSpec((1, BN_ROWS, d), lambda i: (0, i, 0)),
            pl.BlockSpec((1, BN_ROWS, d), lambda i: (1, i, 0)),
            pl.BlockSpec(memory_space=pltpu.SMEM),
            pl.BlockSpec((1, d), lambda i: (0, 0)),
        ],
        out_specs=pl.BlockSpec((BN_ROWS, DOUT), lambda i: (i, 0)),
        out_shape=jax.ShapeDtypeStruct((N, DOUT), jnp.float32),
    )(t, p, p, eps, b)


@jax.jit
def _run(x, edge_index, W0, b0, W1, b1, W2, b2, eps0, eps1, eps2,
         g0, be0, g1, be1):
    npad_e = NW * EWP - E
    pad_src = jnp.arange(npad_e, dtype=jnp.int32) % N
    pad_dst = N + (jnp.arange(npad_e, dtype=jnp.int32) % (NPAD - N))
    src3 = jnp.concatenate([edge_index[0], pad_src]).reshape(
        NW, NGRP, GEC, CHUNK)
    dst3 = jnp.concatenate([edge_index[1], pad_dst]).reshape(
        NW, NGRP, GEC, CHUNK)

    bn_s = 1.0 / jnp.sqrt(1.0 + BN_EPS)
    a0 = (g0 * bn_s).reshape(1, DH)
    a1 = (g1 * bn_s).reshape(1, DH)

    sc_agg = _make_sc_agg(DH)
    sc_agg3 = _make_sc_agg(D3)

    t0 = _tc_matmul(x, W0, DH)
    p0 = sc_agg(t0, src3, dst3)
    t1 = _tc_stage(t0, p0, eps0.reshape(1, 1), b0.reshape(1, DH), a0,
                   be0.reshape(1, DH), W1, DH)
    p1 = sc_agg(t1, src3, dst3)
    t2 = _tc_stage(t1, p1, eps1.reshape(1, 1), b1.reshape(1, DH), a1,
                   be1.reshape(1, DH),
                   jnp.pad(W2, ((0, 0), (0, D3 - DOUT))), D3)
    p2 = sc_agg3(t2, src3, dst3)
    b2p = jnp.pad(b2.reshape(1, DOUT), ((0, 0), (0, D3 - DOUT)))
    return _tc_final(t2, p2, eps2.reshape(1, 1), b2p, D3)


def kernel(x, edge_index, W0, b0, W1, b1, W2, b2, eps0, eps1, eps2,
           g0, be0, g1, be1):
    return _run(x, edge_index, W0, b0, W1, b1, W2, b2, eps0, eps1, eps2,
                g0, be0, g1, be1)


# speedup vs baseline: 1.1200x; 1.0220x over previous
import functools

import jax
import jax.numpy as jnp
from jax import lax
from jax.experimental import pallas as pl
from jax.experimental.pallas import tpu as pltpu
from jax.experimental.pallas import tpu_sc as plsc

N = 10000
E = 320000
DIN = 128
DH = 128
DOUT = 40
D3 = 128
BN_EPS = 1e-5

NC = 2
NS = 16
NW = NC * NS
CHUNK = 128
GEC = 8
NBUF = 2
NPAD = 10240
EWP = NPAD
NCH = EWP // CHUNK
NGRP = NCH // GEC
NSUP = NGRP // 2
RPT = NPAD // NS

ADT = jnp.float32


def _sc_agg_body(d, t_hbm, src_hbm, dst_hbm, out_hbm,
                 sring, dring, bufs, acc, isems, dsems, gsems):
    cid = lax.axis_index("c")
    sid = lax.axis_index("s")
    wid = sid * NC + cid

    lanes = 32 if ADT == jnp.bfloat16 else 16

    def zrow(r, carry):
        for k in range(d // lanes):
            bufs[0][r, pl.ds(k * lanes, lanes)] = jnp.zeros((lanes,), ADT)
        return carry

    lax.fori_loop(0, CHUNK, zrow, 0)
    for i in range(RPT // CHUNK):
        pltpu.sync_copy(bufs[0],
                        acc.at[pl.ds(sid * RPT + i * CHUNK, CHUNK)])
    plsc.subcore_barrier()

    def ldsrc(g, s):
        return pltpu.make_async_copy(src_hbm.at[wid, g], sring[s], isems[s])

    def lddst(g, s):
        return pltpu.make_async_copy(dst_hbm.at[wid, g], dring[s], dsems[s])

    def gth(s, t, b):
        return pltpu.make_async_copy(t_hbm.at[sring[s].at[t]], bufs[b],
                                     gsems[b])

    def sct(s, t, b):
        pltpu.sync_copy(bufs[b], acc.at[dring[s].at[t]], add=True)

    for s in range(2):
        ldsrc(s, s).start()
        lddst(s, s).start()
    ldsrc(0, 0).wait()
    for b in range(NBUF):
        gth(0, b, b).start()
    lddst(0, 0).wait()

    def super_body(k, carry):
        g_next0 = jnp.minimum(2 * k + 2, NGRP - 1)
        g_next1 = jnp.minimum(2 * k + 3, NGRP - 1)
        for half, s in ((0, 0), (1, 1)):
            ns = 1 - s
            for t in range(GEC):
                b = t % NBUF
                gth(s, t, b).wait()
                sct(s, t, b)
                if t < GEC - NBUF:
                    gth(s, t + NBUF, b).start()
                else:
                    if t == GEC - NBUF:
                        ldsrc(0, ns).wait()
                    gth(ns, t - (GEC - NBUF), b).start()
            if half == 0:
                ldsrc(g_next0, 0).start()
                lddst(g_next0, 0).start()
                lddst(1, 1).wait()
        ldsrc(g_next1, 1).start()
        lddst(g_next1, 1).start()
        lddst(0, 0).wait()
        return carry

    lax.fori_loop(0, NSUP, super_body, 0)
    for b in range(NBUF):
        pltpu.make_async_copy(t_hbm.at[sring[0].at[0]], bufs[b],
                              gsems[b]).wait()
    ldsrc(0, 1).wait()
    lddst(0, 1).wait()
    plsc.subcore_barrier()

    pltpu.sync_copy(acc.at[pl.ds(sid * RPT, RPT)],
                    out_hbm.at[cid, pl.ds(sid * RPT, RPT)])


def _make_sc_agg(d):
    mesh = plsc.VectorSubcoreMesh(core_axis_name="c", subcore_axis_name="s",
                                  num_cores=NC, num_subcores=NS)
    return pl.kernel(
        functools.partial(_sc_agg_body, d),
        out_type=jax.ShapeDtypeStruct((NC, NPAD, d), ADT),
        mesh=mesh,
        scratch_types=[
            [pltpu.VMEM((GEC, CHUNK), jnp.int32) for _ in range(2)],
            [pltpu.VMEM((GEC, CHUNK), jnp.int32) for _ in range(2)],
            [pltpu.VMEM((CHUNK, d), ADT) for _ in range(NBUF)],
            pltpu.VMEM_SHARED((NPAD, d), ADT),
            [pltpu.SemaphoreType.DMA for _ in range(2)],
            [pltpu.SemaphoreType.DMA for _ in range(2)],
            [pltpu.SemaphoreType.DMA for _ in range(NBUF)],
        ],
    )


BN_ROWS = 5000


def _mm_body(x_ref, w_ref, o_ref):
    o_ref[...] = jnp.dot(x_ref[...], w_ref[...],
                         preferred_element_type=jnp.float32)


def _tc_matmul(x, w, dout):
    return pl.pallas_call(
        _mm_body,
        grid=(N // BN_ROWS,),
        in_specs=[
            pl.BlockSpec((BN_ROWS, x.shape[1]), lambda i: (i, 0)),
            pl.BlockSpec(w.shape, lambda i: (0, 0)),
        ],
        out_specs=pl.BlockSpec((BN_ROWS, dout), lambda i: (i, 0)),
        out_shape=jax.ShapeDtypeStruct((N, dout), jnp.float32),
    )(x, w)


def _stage_body(t_ref, p0_ref, p1_ref, eps_ref, b_ref, a_ref, be_ref, w_ref,
                o_ref):
    agg = p0_ref[0] + p1_ref[0]
    z = (1.0 + eps_ref[0, 0]) * t_ref[...] + agg + b_ref[...]
    h = jnp.maximum(z * a_ref[...] + be_ref[...], 0.0)
    o_ref[...] = jnp.dot(h, w_ref[...], preferred_element_type=jnp.float32)


def _tc_stage(t, p, eps, b, a, be, w, dout):
    return pl.pallas_call(
        _stage_body,
        grid=(N // BN_ROWS,),
        in_specs=[
            pl.BlockSpec((BN_ROWS, DH), lambda i: (i, 0)),
            pl.BlockSpec((1, BN_ROWS, DH), lambda i: (0, i, 0)),
            pl.BlockSpec((1, BN_ROWS, DH), lambda i: (1, i, 0)),
            pl.BlockSpec(memory_space=pltpu.SMEM),
            pl.BlockSpec((1, DH), lambda i: (0, 0)),
            pl.BlockSpec((1, DH), lambda i: (0, 0)),
            pl.BlockSpec((1, DH), lambda i: (0, 0)),
            pl.BlockSpec((DH, dout), lambda i: (0, 0)),
        ],
        out_specs=pl.BlockSpec((BN_ROWS, dout), lambda i: (i, 0)),
        out_shape=jax.ShapeDtypeStruct((N, dout), jnp.float32),
    )(t, p, p, eps, b, a, be, w)


def _final_body(t_ref, p0_ref, p1_ref, eps_ref, b_ref, o_ref):
    agg = p0_ref[0] + p1_ref[0]
    z = (1.0 + eps_ref[0, 0]) * t_ref[...] + agg + b_ref[...]
    mask = lax.broadcasted_iota(jnp.int32, z.shape, 1) < DOUT
    zm = jnp.where(mask, z, -jnp.inf)
    m = jnp.max(zm, axis=-1, keepdims=True)
    ez = jnp.where(mask, jnp.exp(z - m), 0.0)
    ls = (z - m) - jnp.log(jnp.sum(ez, axis=-1, keepdims=True))
    o_ref[...] = ls[:, :DOUT]


def _tc_final(t, p, eps, b, d):
    return pl.pallas_call(
        _final_body,
        grid=(N // BN_ROWS,),
        in_specs=[
            pl.BlockSpec((BN_ROWS, d), lambda i: (i, 0)),
            pl.BlockSpec((1, BN_ROWS, d), lambda i: (0, i, 0)),
            pl.BlockSpec((1, BN_ROWS, d), lambda i: (1, i, 0)),
            pl.BlockSpec(memory_space=pltpu.SMEM),
            pl.BlockSpec((1, d), lambda i: (0, 0)),
        ],
        out_specs=pl.BlockSpec((BN_ROWS, DOUT), lambda i: (i, 0)),
        out_shape=jax.ShapeDtypeStruct((N, DOUT), jnp.float32),
    )(t, p, p, eps, b)


@jax.jit
def _run(x, edge_index, W0, b0, W1, b1, W2, b2, eps0, eps1, eps2,
         g0, be0, g1, be1):
    npad_e = NW * EWP - E
    pad_src = jnp.arange(npad_e, dtype=jnp.int32) % N
    pad_dst = N + (jnp.arange(npad_e, dtype=jnp.int32) % (NPAD - N))
    src3 = jnp.concatenate([edge_index[0], pad_src]).reshape(
        NW, NGRP, GEC, CHUNK)
    dst3 = jnp.concatenate([edge_index[1], pad_dst]).reshape(
        NW, NGRP, GEC, CHUNK)

    bn_s = 1.0 / jnp.sqrt(1.0 + BN_EPS)
    a0 = (g0 * bn_s).reshape(1, DH)
    a1 = (g1 * bn_s).reshape(1, DH)

    sc_agg = _make_sc_agg(DH)
    sc_agg3 = _make_sc_agg(D3)

    t0 = _tc_matmul(x, W0, DH)
    p0 = sc_agg(t0, src3, dst3)
    t1 = _tc_stage(t0, p0, eps0.reshape(1, 1), b0.reshape(1, DH), a0,
                   be0.reshape(1, DH), W1, DH)
    p1 = sc_agg(t1, src3, dst3)
    t2 = _tc_stage(t1, p1, eps1.reshape(1, 1), b1.reshape(1, DH), a1,
                   be1.reshape(1, DH),
                   jnp.pad(W2, ((0, 0), (0, D3 - DOUT))), D3)
    p2 = sc_agg3(t2, src3, dst3)
    b2p = jnp.pad(b2.reshape(1, DOUT), ((0, 0), (0, D3 - DOUT)))
    return _tc_final(t2, p2, eps2.reshape(1, 1), b2p, D3)


def kernel(x, edge_index, W0, b0, W1, b1, W2, b2, eps0, eps1, eps2,
           g0, be0, g1, be1):
    return _run(x, edge_index, W0, b0, W1, b1, W2, b2, eps0, eps1, eps2,
                g0, be0, g1, be1)
